# Initial kernel scaffold; baseline (speedup 1.0000x reference)
import functools

import jax
import jax.numpy as jnp
from jax import lax
from jax.experimental import pallas as pl
from jax.experimental.pallas import tpu as pltpu
from jax.experimental.pallas import tpu_sc as plsc

N_NODES = 50000
DIM = 64
DP = 128
N_EDGES = 800000

NC = 2
NS = 16
NQ = 4
QN = N_NODES // NQ
QROWS = 12512
QPT = QROWS // NS
TRASH = QN
ZC = 391

SUB = 125
ROWS = N_EDGES // SUB
RPT = ROWS // NS
GRP = 4
NGRP = RPT // GRP

E_PAD = 800256
EPT = E_PAD // (NC * NS)
HIST = 51200
HSL = HIST // NS

_mesh = plsc.VectorSubcoreMesh(core_axis_name="c", subcore_axis_name="s")
_cp = pltpu.CompilerParams(needs_layout_passes=False)


@functools.partial(
    pl.kernel,
    out_type=jax.ShapeDtypeStruct((NC, HIST), jnp.float32),
    mesh=_mesh,
    scratch_types=[
        pltpu.VMEM_SHARED((NS, HIST), jnp.float32),
        pltpu.VMEM((EPT,), jnp.int32),
        pltpu.VMEM((HIST,), jnp.float32),
        pltpu.VMEM((HSL,), jnp.float32),
        pltpu.VMEM((HSL,), jnp.float32),
    ],
    compiler_params=_cp,
)
def _deg_kernel(dst_hbm, hist_hbm, hist_sp, idx_v, hist_v, tmp_v, acc_v):
    c = lax.axis_index("c")
    s = lax.axis_index("s")
    w = s * NC + c

    pltpu.sync_copy(dst_hbm.at[pl.ds(w * EPT, EPT)], idx_v)

    def zero_body(i, _):
        hist_v[pl.ds(i * 16, 16)] = jnp.zeros((16,), jnp.float32)
        return 0
    lax.fori_loop(0, HIST // 16, zero_body, 0)

    ones16 = jnp.ones((16,), jnp.float32)

    def scat_body(i, _):
        idx16 = idx_v[pl.ds(i * 16, 16)]
        plsc.addupdate_scatter(hist_v, [idx16], ones16)
        return 0
    lax.fori_loop(0, EPT // 16, scat_body, 0)

    pltpu.sync_copy(hist_v, hist_sp.at[s])
    plsc.subcore_barrier()

    pltpu.sync_copy(hist_sp.at[0, pl.ds(s * HSL, HSL)], acc_v)
    for k in range(1, NS):
        pltpu.sync_copy(hist_sp.at[k, pl.ds(s * HSL, HSL)], tmp_v)

        def add_body(i, _):
            sl = pl.ds(i * 16, 16)
            acc_v[sl] = acc_v[sl] + tmp_v[sl]
            return 0
        lax.fori_loop(0, HSL // 16, add_body, 0)

    pltpu.sync_copy(acc_v, hist_hbm.at[c, pl.ds(s * HSL, HSL)])


@functools.partial(
    pl.kernel,
    out_type=jax.ShapeDtypeStruct((NQ, QROWS, DP), jnp.float32),
    mesh=_mesh,
    scratch_types=[
        pltpu.VMEM_SHARED((QROWS, DP), jnp.float32),
        pltpu.VMEM((GRP, SUB), jnp.int32),
        pltpu.VMEM((GRP, SUB), jnp.int32),
        pltpu.VMEM((GRP, SUB, DP), jnp.float32),
        pltpu.SemaphoreType.DMA,
        pltpu.SemaphoreType.DMA,
    ],
    compiler_params=_cp,
)
def _layer_kernel(src_hbm, dstq_hbm, z_hbm, zero_hbm, t_hbm,
                  acc_sp, src_v, dst_v, rows_v, sem_g, sem_s):
    c = lax.axis_index("c")
    s = lax.axis_index("s")

    for p in range(2):
        q = 2 * c + p

        for k in range(QPT // ZC):
            pltpu.sync_copy(zero_hbm,
                            acc_sp.at[pl.ds(s * QPT + k * ZC, ZC)])
        plsc.subcore_barrier()

        def group_body(g, _):
            row0 = s * RPT + g * GRP
            pltpu.sync_copy(src_hbm.at[pl.ds(row0, GRP)], src_v)
            pltpu.sync_copy(dstq_hbm.at[q, pl.ds(row0, GRP)], dst_v)
            for j in range(GRP):
                pltpu.async_copy(z_hbm.at[src_v.at[j]], rows_v.at[j], sem_g)
            for j in range(GRP):
                pltpu.make_async_copy(
                    z_hbm.at[src_v.at[0]], rows_v.at[0], sem_g).wait()
            for j in range(GRP):
                pltpu.async_copy(rows_v.at[j], acc_sp.at[dst_v.at[j]],
                                 sem_s, add=True)
            for j in range(GRP):
                pltpu.make_async_copy(
                    rows_v.at[0], acc_sp.at[dst_v.at[0]], sem_s).wait()
            return 0

        lax.fori_loop(0, NGRP, group_body, 0)
        plsc.subcore_barrier()

        for k in range(QPT // ZC):
            off = s * QPT + k * ZC
            pltpu.sync_copy(acc_sp.at[pl.ds(off, ZC)],
                            t_hbm.at[q, pl.ds(off, ZC)])
        plsc.subcore_barrier()


_BN = 2000


def _finalize_body(hist_ref, emb_ref, dinv_ref, z0_ref):
    deg = hist_ref[:, 0:1] + hist_ref[:, 1:2]
    safe = jnp.where(deg > 0, deg, 1.0)
    dinv = jnp.where(deg > 0, lax.rsqrt(safe), 0.0)
    dinv_rep = jnp.broadcast_to(dinv, (_BN, DIM))
    dinv_ref[...] = dinv_rep
    z0_ref[...] = jnp.concatenate(
        [dinv_rep * emb_ref[...], jnp.zeros((_BN, DIM), jnp.float32)], axis=1)


def _tc_finalize(hist_t, emb):
    return pl.pallas_call(
        _finalize_body,
        grid=(N_NODES // _BN,),
        in_specs=[
            pl.BlockSpec((_BN, 2), lambda i: (i, 0)),
            pl.BlockSpec((_BN, DIM), lambda i: (i, 0)),
        ],
        out_specs=[
            pl.BlockSpec((_BN, DIM), lambda i: (i, 0)),
            pl.BlockSpec((_BN, DP), lambda i: (i, 0)),
        ],
        out_shape=[
            jax.ShapeDtypeStruct((N_NODES, DIM), jnp.float32),
            jax.ShapeDtypeStruct((N_NODES, DP), jnp.float32),
        ],
    )(hist_t, emb)


def _scale_body(t_ref, dinv_ref, z_ref):
    d = dinv_ref[...]
    z_ref[...] = jnp.concatenate(
        [d * d * t_ref[...], jnp.zeros((_BN, DIM), jnp.float32)], axis=1)


def _tc_scale(t_cat, dinv_rep):
    return pl.pallas_call(
        _scale_body,
        grid=(N_NODES // _BN,),
        in_specs=[
            pl.BlockSpec((_BN, DIM), lambda i: (i, 0)),
            pl.BlockSpec((_BN, DIM), lambda i: (i, 0)),
        ],
        out_specs=pl.BlockSpec((_BN, DP), lambda i: (i, 0)),
        out_shape=jax.ShapeDtypeStruct((N_NODES, DP), jnp.float32),
    )(t_cat, dinv_rep)


def _final_body(t0_ref, t1_ref, t2_ref, dinv_ref, emb_ref, out_ref):
    tsum = t0_ref[...] + t1_ref[...] + t2_ref[...]
    out_ref[...] = (emb_ref[...] + dinv_ref[...] * tsum) * 0.25


def _tc_final(t0, t1, t2, dinv_rep, emb):
    spec = pl.BlockSpec((_BN, DIM), lambda i: (i, 0))
    return pl.pallas_call(
        _final_body,
        grid=(N_NODES // _BN,),
        in_specs=[spec] * 5,
        out_specs=spec,
        out_shape=jax.ShapeDtypeStruct((N_NODES, DIM), jnp.float32),
    )(t0, t1, t2, dinv_rep, emb)


def _t_cat(t4):
    return t4[:, :QN, :DIM].reshape(N_NODES, DIM)


def kernel(edge_index, emb_weight):
    src = edge_index[0].astype(jnp.int32)
    dst = edge_index[1].astype(jnp.int32)

    dst_pad = jnp.concatenate(
        [dst, jnp.full((E_PAD - N_EDGES,), N_NODES, jnp.int32)])
    src_r = src.reshape(ROWS, SUB)
    qs = jnp.arange(NQ, dtype=jnp.int32)[:, None] * QN
    local = dst[None, :] - qs
    dstq = jnp.where((local >= 0) & (local < QN), local,
                     jnp.int32(TRASH)).reshape(NQ, ROWS, SUB)
    zero_tile = jnp.zeros((ZC, DP), jnp.float32)

    hist = _deg_kernel(dst_pad)
    hist_t = hist.T[:N_NODES]
    dinv_rep, z = _tc_finalize(hist_t, emb_weight)

    t0 = _layer_kernel(src_r, dstq, z, zero_tile)
    t0c = _t_cat(t0)
    z = _tc_scale(t0c, dinv_rep)
    t1 = _layer_kernel(src_r, dstq, z, zero_tile)
    t1c = _t_cat(t1)
    z = _tc_scale(t1c, dinv_rep)
    t2 = _layer_kernel(src_r, dstq, z, zero_tile)

    out = _tc_final(t0c, t1c, _t_cat(t2), dinv_rep, emb_weight)
    return (emb_weight, out)


# speedup vs baseline: 3.8298x; 3.8298x over previous
import functools

import jax
import jax.numpy as jnp
from jax import lax
from jax.experimental import pallas as pl
from jax.experimental.pallas import tpu as pltpu
from jax.experimental.pallas import tpu_sc as plsc

N_NODES = 50000
DIM = 64
DP = 128
N_EDGES = 800000

NC = 2
NS = 16
NQ = 4
QN = N_NODES // NQ
QROWS = 12544
QPT = QROWS // NS
TRASH = QN
ZC = 392

SUB = 50
ROWS = N_EDGES // SUB
RPT = ROWS // NS
GRP = 4
CH8 = 8
NCH = RPT // CH8

E_PAD = 800256
EPT = E_PAD // (NC * NS)
ECH = 8336
HIST = 51200
NW = NC * NS

_mesh = plsc.VectorSubcoreMesh(core_axis_name="c", subcore_axis_name="s")
_cp = pltpu.CompilerParams(needs_layout_passes=False)


@functools.partial(
    pl.kernel,
    out_type=jax.ShapeDtypeStruct((NW, HIST), jnp.float32),
    mesh=_mesh,
    scratch_types=[
        pltpu.VMEM((ECH,), jnp.int32),
        pltpu.VMEM((HIST,), jnp.float32),
    ],
    compiler_params=_cp,
)
def _deg_kernel(dst_hbm, hist_hbm, idx_v, hist_v):
    c = lax.axis_index("c")
    s = lax.axis_index("s")
    w = s * NC + c

    def zero_body(i, _):
        hist_v[pl.ds(i * 16, 16)] = jnp.zeros((16,), jnp.float32)
        return 0
    lax.fori_loop(0, HIST // 16, zero_body, 0)

    ones16 = jnp.ones((16,), jnp.float32)

    def chunk_body(k, _):
        pltpu.sync_copy(dst_hbm.at[pl.ds(w * EPT + k * ECH, ECH)], idx_v)

        def scat_body(i, _2):
            idx16 = idx_v[pl.ds(i * 16, 16)]
            plsc.addupdate_scatter(hist_v, [idx16], ones16)
            return 0
        lax.fori_loop(0, ECH // 16, scat_body, 0)
        return 0
    lax.fori_loop(0, EPT // ECH, chunk_body, 0)

    pltpu.sync_copy(hist_v, hist_hbm.at[w])


@functools.partial(
    pl.kernel,
    out_type=jax.ShapeDtypeStruct((NQ, QROWS, DP), jnp.float32),
    mesh=_mesh,
    scratch_types=[
        pltpu.VMEM_SHARED((QROWS, DP), jnp.float32),
        pltpu.VMEM((CH8, SUB), jnp.int32),
        pltpu.VMEM((CH8, SUB), jnp.int32),
        pltpu.VMEM((GRP, SUB, DP), jnp.float32),
        pltpu.SemaphoreType.DMA,
        pltpu.SemaphoreType.DMA,
    ],
    compiler_params=_cp,
)
def _layer_kernel(src_hbm, dstq_hbm, z_hbm, zero_hbm, t_hbm,
                  acc_sp, src_v, dst_v, rows_v, sem_g, sem_s):
    c = lax.axis_index("c")
    s = lax.axis_index("s")

    for p in range(2):
        q = 2 * c + p

        for k in range(QPT // ZC):
            pltpu.sync_copy(zero_hbm,
                            acc_sp.at[pl.ds(s * QPT + k * ZC, ZC)])
        plsc.subcore_barrier()

        def chunk_body(ch, _):
            row0 = s * RPT + ch * CH8
            pltpu.sync_copy(src_hbm.at[pl.ds(row0, CH8)], src_v)
            pltpu.sync_copy(dstq_hbm.at[q, pl.ds(row0, CH8)], dst_v)
            for h in range(CH8 // GRP):
                for j in range(GRP):
                    pltpu.async_copy(z_hbm.at[src_v.at[h * GRP + j]],
                                     rows_v.at[j], sem_g)
                for j in range(GRP):
                    pltpu.make_async_copy(
                        z_hbm.at[src_v.at[0]], rows_v.at[0], sem_g).wait()
                for j in range(GRP):
                    pltpu.async_copy(rows_v.at[j],
                                     acc_sp.at[dst_v.at[h * GRP + j]],
                                     sem_s, add=True)
                for j in range(GRP):
                    pltpu.make_async_copy(
                        rows_v.at[0], acc_sp.at[dst_v.at[0]], sem_s).wait()
            return 0

        lax.fori_loop(0, NCH, chunk_body, 0)
        plsc.subcore_barrier()

        for k in range(QPT // ZC):
            off = s * QPT + k * ZC
            pltpu.sync_copy(acc_sp.at[pl.ds(off, ZC)],
                            t_hbm.at[q, pl.ds(off, ZC)])
        plsc.subcore_barrier()


_BN = 2000


def _finalize_body(hist_ref, emb_ref, dinv_ref, z0_ref):
    deg = jnp.sum(hist_ref[...], axis=1, keepdims=True)
    safe = jnp.where(deg > 0, deg, 1.0)
    dinv = jnp.where(deg > 0, lax.rsqrt(safe), 0.0)
    dinv_rep = jnp.broadcast_to(dinv, (_BN, DIM))
    dinv_ref[...] = dinv_rep
    z0_ref[...] = jnp.concatenate(
        [dinv_rep * emb_ref[...], jnp.zeros((_BN, DIM), jnp.float32)], axis=1)


def _tc_finalize(hist_t, emb):
    return pl.pallas_call(
        _finalize_body,
        grid=(N_NODES // _BN,),
        in_specs=[
            pl.BlockSpec((_BN, NW), lambda i: (i, 0)),
            pl.BlockSpec((_BN, DIM), lambda i: (i, 0)),
        ],
        out_specs=[
            pl.BlockSpec((_BN, DIM), lambda i: (i, 0)),
            pl.BlockSpec((_BN, DP), lambda i: (i, 0)),
        ],
        out_shape=[
            jax.ShapeDtypeStruct((N_NODES, DIM), jnp.float32),
            jax.ShapeDtypeStruct((N_NODES, DP), jnp.float32),
        ],
    )(hist_t, emb)


def _scale_body(t_ref, dinv_ref, z_ref):
    d = dinv_ref[...]
    z_ref[...] = jnp.concatenate(
        [d * d * t_ref[...], jnp.zeros((_BN, DIM), jnp.float32)], axis=1)


def _tc_scale(t_cat, dinv_rep):
    return pl.pallas_call(
        _scale_body,
        grid=(N_NODES // _BN,),
        in_specs=[
            pl.BlockSpec((_BN, DIM), lambda i: (i, 0)),
            pl.BlockSpec((_BN, DIM), lambda i: (i, 0)),
        ],
        out_specs=pl.BlockSpec((_BN, DP), lambda i: (i, 0)),
        out_shape=jax.ShapeDtypeStruct((N_NODES, DP), jnp.float32),
    )(t_cat, dinv_rep)


def _final_body(t0_ref, t1_ref, t2_ref, dinv_ref, emb_ref, out_ref):
    tsum = t0_ref[...] + t1_ref[...] + t2_ref[...]
    out_ref[...] = (emb_ref[...] + dinv_ref[...] * tsum) * 0.25


def _tc_final(t0, t1, t2, dinv_rep, emb):
    spec = pl.BlockSpec((_BN, DIM), lambda i: (i, 0))
    return pl.pallas_call(
        _final_body,
        grid=(N_NODES // _BN,),
        in_specs=[spec] * 5,
        out_specs=spec,
        out_shape=jax.ShapeDtypeStruct((N_NODES, DIM), jnp.float32),
    )(t0, t1, t2, dinv_rep, emb)


def _t_cat(t4):
    return t4[:, :QN, :DIM].reshape(N_NODES, DIM)


def kernel(edge_index, emb_weight):
    src = edge_index[0].astype(jnp.int32)
    dst = edge_index[1].astype(jnp.int32)

    dst_pad = jnp.concatenate(
        [dst, jnp.full((E_PAD - N_EDGES,), N_NODES, jnp.int32)])
    src_r = src.reshape(ROWS, SUB)
    qs = jnp.arange(NQ, dtype=jnp.int32)[:, None] * QN
    local = dst[None, :] - qs
    dstq = jnp.where((local >= 0) & (local < QN), local,
                     jnp.int32(TRASH)).reshape(NQ, ROWS, SUB)
    zero_tile = jnp.zeros((ZC, DP), jnp.float32)

    hist = _deg_kernel(dst_pad)
    hist_t = hist.T[:N_NODES]
    dinv_rep, z = _tc_finalize(hist_t, emb_weight)

    t0 = _layer_kernel(src_r, dstq, z, zero_tile)
    t0c = _t_cat(t0)
    z = _tc_scale(t0c, dinv_rep)
    t1 = _layer_kernel(src_r, dstq, z, zero_tile)
    t1c = _t_cat(t1)
    z = _tc_scale(t1c, dinv_rep)
    t2 = _layer_kernel(src_r, dstq, z, zero_tile)

    out = _tc_final(t0c, t1c, _t_cat(t2), dinv_rep, emb_weight)
    return (emb_weight, out)


# speedup vs baseline: 7.2172x; 1.8845x over previous
import functools

import jax
import jax.numpy as jnp
from jax import lax
from jax.experimental import pallas as pl
from jax.experimental.pallas import tpu as pltpu
from jax.experimental.pallas import tpu_sc as plsc

N_NODES = 50000
DIM = 64
DP = 128
N_EDGES = 800000

NC = 2
NS = 16
NH = 2
HN = N_NODES // NH
QN = HN // 2
QROWS = 12544
QPT = QROWS // NS
TRASH = QN
ZC = 392

SUB = 50
ROWS = N_EDGES // SUB
RPT = ROWS // NS
GRP = 4
CH8 = 8
NCH = RPT // CH8

E_PAD = 800256
EPT = E_PAD // (NC * NS)
ECH = 8336
HIST = 51200
NW = NC * NS

_mesh = plsc.VectorSubcoreMesh(core_axis_name="c", subcore_axis_name="s")
_cp = pltpu.CompilerParams(needs_layout_passes=False)


@functools.partial(
    pl.kernel,
    out_type=jax.ShapeDtypeStruct((NW, HIST), jnp.float32),
    mesh=_mesh,
    scratch_types=[
        pltpu.VMEM((ECH,), jnp.int32),
        pltpu.VMEM((HIST,), jnp.float32),
    ],
    compiler_params=_cp,
)
def _deg_kernel(dst_hbm, hist_hbm, idx_v, hist_v):
    c = lax.axis_index("c")
    s = lax.axis_index("s")
    w = s * NC + c

    def zero_body(i, _):
        hist_v[pl.ds(i * 16, 16)] = jnp.zeros((16,), jnp.float32)
        return 0
    lax.fori_loop(0, HIST // 16, zero_body, 0)

    ones16 = jnp.ones((16,), jnp.float32)

    def chunk_body(k, _):
        pltpu.sync_copy(dst_hbm.at[pl.ds(w * EPT + k * ECH, ECH)], idx_v)

        def scat_body(i, _2):
            idx16 = idx_v[pl.ds(i * 16, 16)]
            plsc.addupdate_scatter(hist_v, [idx16], ones16)
            return 0
        lax.fori_loop(0, ECH // 16, scat_body, 0)
        return 0
    lax.fori_loop(0, EPT // ECH, chunk_body, 0)

    pltpu.sync_copy(hist_v, hist_hbm.at[w])


@functools.partial(
    pl.kernel,
    out_type=jax.ShapeDtypeStruct((NH, QROWS, DP), jnp.float32),
    mesh=_mesh,
    scratch_types=[
        pltpu.VMEM_SHARED((QROWS, DP), jnp.float32),
        pltpu.VMEM((CH8, SUB), jnp.int32),
        pltpu.VMEM((CH8, SUB), jnp.int32),
        pltpu.VMEM((GRP, SUB, DP), jnp.float32),
        pltpu.SemaphoreType.DMA,
        pltpu.SemaphoreType.DMA,
    ],
    compiler_params=_cp,
)
def _layer_kernel(src_hbm, dstq_hbm, z_hbm, zero_hbm, t_hbm,
                  acc_sp, src_v, dst_v, rows_v, sem_g, sem_s):
    c = lax.axis_index("c")
    s = lax.axis_index("s")

    for k in range(QPT // ZC):
        pltpu.sync_copy(zero_hbm,
                        acc_sp.at[pl.ds(s * QPT + k * ZC, ZC)])
    plsc.subcore_barrier()

    def chunk_body(ch, _):
        row0 = s * RPT + ch * CH8
        pltpu.sync_copy(src_hbm.at[pl.ds(row0, CH8)], src_v)
        pltpu.sync_copy(dstq_hbm.at[c, pl.ds(row0, CH8)], dst_v)
        for h in range(CH8 // GRP):
            for j in range(GRP):
                pltpu.async_copy(z_hbm.at[src_v.at[h * GRP + j]],
                                 rows_v.at[j], sem_g)
            for j in range(GRP):
                pltpu.make_async_copy(
                    z_hbm.at[src_v.at[0]], rows_v.at[0], sem_g).wait()
            for j in range(GRP):
                pltpu.async_copy(rows_v.at[j],
                                 acc_sp.at[dst_v.at[h * GRP + j]],
                                 sem_s, add=True)
            for j in range(GRP):
                pltpu.make_async_copy(
                    rows_v.at[0], acc_sp.at[dst_v.at[0]], sem_s).wait()
        return 0

    lax.fori_loop(0, NCH, chunk_body, 0)
    plsc.subcore_barrier()

    for k in range(QPT // ZC):
        off = s * QPT + k * ZC
        pltpu.sync_copy(acc_sp.at[pl.ds(off, ZC)],
                        t_hbm.at[c, pl.ds(off, ZC)])


_BN = 2000


def _z2_block(z, j):
    zero = jnp.zeros((_BN, DIM), jnp.float32)
    top = jnp.concatenate([z, zero], axis=1)
    bot = jnp.concatenate([zero, z], axis=1)
    return jnp.where(j == 0, top, bot)[None]


def _finalize_body(hist_ref, emb_ref, dinv_ref, z0_ref):
    j = pl.program_id(0)
    deg = jnp.sum(hist_ref[...], axis=1, keepdims=True)
    safe = jnp.where(deg > 0, deg, 1.0)
    dinv = jnp.where(deg > 0, lax.rsqrt(safe), 0.0)
    dinv_rep = jnp.broadcast_to(dinv, (_BN, DIM))
    dinv_ref[...] = dinv_rep
    z0_ref[...] = _z2_block(dinv_rep * emb_ref[...], j)


def _tc_finalize(hist_t, emb):
    return pl.pallas_call(
        _finalize_body,
        grid=(2, N_NODES // _BN),
        in_specs=[
            pl.BlockSpec((_BN, NW), lambda j, i: (i, 0)),
            pl.BlockSpec((_BN, DIM), lambda j, i: (i, 0)),
        ],
        out_specs=[
            pl.BlockSpec((_BN, DIM), lambda j, i: (i, 0)),
            pl.BlockSpec((1, _BN, DP), lambda j, i: (j, i, 0)),
        ],
        out_shape=[
            jax.ShapeDtypeStruct((N_NODES, DIM), jnp.float32),
            jax.ShapeDtypeStruct((2, N_NODES, DP), jnp.float32),
        ],
    )(hist_t, emb)


def _scale_body(t_ref, dinv_ref, z_ref):
    j = pl.program_id(0)
    d = dinv_ref[...]
    z_ref[...] = _z2_block(d * d * t_ref[...], j)


def _tc_scale(t_cat, dinv_rep):
    return pl.pallas_call(
        _scale_body,
        grid=(2, N_NODES // _BN),
        in_specs=[
            pl.BlockSpec((_BN, DIM), lambda j, i: (i, 0)),
            pl.BlockSpec((_BN, DIM), lambda j, i: (i, 0)),
        ],
        out_specs=pl.BlockSpec((1, _BN, DP), lambda j, i: (j, i, 0)),
        out_shape=jax.ShapeDtypeStruct((2, N_NODES, DP), jnp.float32),
    )(t_cat, dinv_rep)


def _final_body(t0_ref, t1_ref, t2_ref, dinv_ref, emb_ref, out_ref):
    tsum = t0_ref[...] + t1_ref[...] + t2_ref[...]
    out_ref[...] = (emb_ref[...] + dinv_ref[...] * tsum) * 0.25


def _tc_final(t0, t1, t2, dinv_rep, emb):
    spec = pl.BlockSpec((_BN, DIM), lambda i: (i, 0))
    return pl.pallas_call(
        _final_body,
        grid=(N_NODES // _BN,),
        in_specs=[spec] * 5,
        out_specs=spec,
        out_shape=jax.ShapeDtypeStruct((N_NODES, DIM), jnp.float32),
    )(t0, t1, t2, dinv_rep, emb)


def _t_cat(t2):
    return t2[:, :QN, :].reshape(NH, QN * 2, DIM).reshape(N_NODES, DIM)


def kernel(edge_index, emb_weight):
    src = edge_index[0].astype(jnp.int32)
    dst = edge_index[1].astype(jnp.int32)

    dst_pad = jnp.concatenate(
        [dst, jnp.full((E_PAD - N_EDGES,), N_NODES, jnp.int32)])
    src_r = ((dst & 1) * N_NODES + src).reshape(ROWS, SUB)
    hs = jnp.arange(NH, dtype=jnp.int32)[:, None] * HN
    local = dst[None, :] - hs
    dstq = jnp.where((local >= 0) & (local < HN), local >> 1,
                     jnp.int32(TRASH)).reshape(NH, ROWS, SUB)
    zero_tile = jnp.zeros((ZC, DP), jnp.float32)

    hist = _deg_kernel(dst_pad)
    hist_t = hist.T[:N_NODES]
    dinv_rep, z = _tc_finalize(hist_t, emb_weight)

    zf = z.reshape(2 * N_NODES, DP)
    t0 = _layer_kernel(src_r, dstq, zf, zero_tile)
    t0c = _t_cat(t0)
    zf = _tc_scale(t0c, dinv_rep).reshape(2 * N_NODES, DP)
    t1 = _layer_kernel(src_r, dstq, zf, zero_tile)
    t1c = _t_cat(t1)
    zf = _tc_scale(t1c, dinv_rep).reshape(2 * N_NODES, DP)
    t2 = _layer_kernel(src_r, dstq, zf, zero_tile)

    out = _tc_final(t0c, t1c, _t_cat(t2), dinv_rep, emb_weight)
    return (emb_weight, out)


# speedup vs baseline: 7.6944x; 1.0661x over previous
import functools

import jax
import jax.numpy as jnp
from jax import lax
from jax.experimental import pallas as pl
from jax.experimental.pallas import tpu as pltpu
from jax.experimental.pallas import tpu_sc as plsc

N_NODES = 50000
DIM = 64
DP = 128
N_EDGES = 800000

NC = 2
NS = 16
NH = 2
HN = N_NODES // NH
QN = HN // 2
QROWS = 12544
QPT = QROWS // NS
TRASH = QN
ZC = 392

SUB = 50
ROWS = N_EDGES // SUB
RPT = ROWS // NS
GRP = 4
CH8 = 8
NCH = RPT // CH8

E_PAD = 800256
EPT = E_PAD // (NC * NS)
ECH = 8336
HIST = 51200
NW = NC * NS

_mesh = plsc.VectorSubcoreMesh(core_axis_name="c", subcore_axis_name="s")
_cp = pltpu.CompilerParams(needs_layout_passes=False)


@functools.partial(
    pl.kernel,
    out_type=jax.ShapeDtypeStruct((NW, HIST), jnp.float32),
    mesh=_mesh,
    scratch_types=[
        pltpu.VMEM((ECH,), jnp.int32),
        pltpu.VMEM((HIST,), jnp.float32),
    ],
    compiler_params=_cp,
)
def _deg_kernel(dst_hbm, hist_hbm, idx_v, hist_v):
    c = lax.axis_index("c")
    s = lax.axis_index("s")
    w = s * NC + c

    def zero_body(i, _):
        hist_v[pl.ds(i * 16, 16)] = jnp.zeros((16,), jnp.float32)
        return 0
    lax.fori_loop(0, HIST // 16, zero_body, 0)

    ones16 = jnp.ones((16,), jnp.float32)

    def chunk_body(k, _):
        pltpu.sync_copy(dst_hbm.at[pl.ds(w * EPT + k * ECH, ECH)], idx_v)

        def scat_body(i, _2):
            idx16 = idx_v[pl.ds(i * 16, 16)]
            plsc.addupdate_scatter(hist_v, [idx16], ones16)
            return 0
        lax.fori_loop(0, ECH // 16, scat_body, 0)
        return 0
    lax.fori_loop(0, EPT // ECH, chunk_body, 0)

    pltpu.sync_copy(hist_v, hist_hbm.at[w])


@functools.partial(
    pl.kernel,
    out_type=jax.ShapeDtypeStruct((NH, QROWS, DP), jnp.float32),
    mesh=_mesh,
    scratch_types=[
        pltpu.VMEM_SHARED((QROWS, DP), jnp.float32),
        pltpu.VMEM((CH8, SUB), jnp.int32),
        pltpu.VMEM((CH8, SUB), jnp.int32),
        pltpu.VMEM((GRP, SUB, DP), jnp.float32),
        pltpu.SemaphoreType.DMA,
        pltpu.SemaphoreType.DMA,
        pltpu.SemaphoreType.DMA,
        pltpu.SemaphoreType.DMA,
    ],
    compiler_params=_cp,
)
def _layer_kernel(src_hbm, dstq_hbm, z_hbm, zero_hbm, t_hbm,
                  acc_sp, src_v, dst_v, rows_v, sem_g0, sem_g1,
                  sem_s0, sem_s1):
    c = lax.axis_index("c")
    s = lax.axis_index("s")

    for k in range(QPT // ZC):
        pltpu.sync_copy(zero_hbm,
                        acc_sp.at[pl.ds(s * QPT + k * ZC, ZC)])
    plsc.subcore_barrier()

    def chunk_body(ch, _):
        row0 = s * RPT + ch * CH8
        pltpu.sync_copy(src_hbm.at[pl.ds(row0, CH8)], src_v)
        pltpu.sync_copy(dstq_hbm.at[c, pl.ds(row0, CH8)], dst_v)

        def fg(buf, r, sem):
            for j in range(2):
                pltpu.async_copy(z_hbm.at[src_v.at[r + j]],
                                 rows_v.at[2 * buf + j], sem)

        def dg(sem):
            for j in range(2):
                pltpu.make_async_copy(
                    z_hbm.at[src_v.at[0]], rows_v.at[0], sem).wait()

        def fs(buf, r, sem):
            for j in range(2):
                pltpu.async_copy(rows_v.at[2 * buf + j],
                                 acc_sp.at[dst_v.at[r + j]], sem, add=True)

        def ds(sem):
            for j in range(2):
                pltpu.make_async_copy(
                    rows_v.at[0], acc_sp.at[dst_v.at[0]], sem).wait()

        fg(0, 0, sem_g0)
        fg(1, 2, sem_g1)
        dg(sem_g0); fs(0, 0, sem_s0)
        ds(sem_s0); fg(0, 4, sem_g0)
        dg(sem_g1); fs(1, 2, sem_s1)
        ds(sem_s1); fg(1, 6, sem_g1)
        dg(sem_g0); fs(0, 4, sem_s0); ds(sem_s0)
        dg(sem_g1); fs(1, 6, sem_s1); ds(sem_s1)
        return 0

    lax.fori_loop(0, NCH, chunk_body, 0)
    plsc.subcore_barrier()

    for k in range(QPT // ZC):
        off = s * QPT + k * ZC
        pltpu.sync_copy(acc_sp.at[pl.ds(off, ZC)],
                        t_hbm.at[c, pl.ds(off, ZC)])


_BN = 2000


def _z2_block(z, j):
    zero = jnp.zeros((_BN, DIM), jnp.float32)
    top = jnp.concatenate([z, zero], axis=1)
    bot = jnp.concatenate([zero, z], axis=1)
    return jnp.where(j == 0, top, bot)[None]


def _finalize_body(hist_ref, emb_ref, dinv_ref, z0_ref):
    j = pl.program_id(0)
    deg = jnp.sum(hist_ref[...], axis=1, keepdims=True)
    safe = jnp.where(deg > 0, deg, 1.0)
    dinv = jnp.where(deg > 0, lax.rsqrt(safe), 0.0)
    dinv_rep = jnp.broadcast_to(dinv, (_BN, DIM))
    dinv_ref[...] = dinv_rep
    z0_ref[...] = _z2_block(dinv_rep * emb_ref[...], j)


def _tc_finalize(hist_t, emb):
    return pl.pallas_call(
        _finalize_body,
        grid=(2, N_NODES // _BN),
        in_specs=[
            pl.BlockSpec((_BN, NW), lambda j, i: (i, 0)),
            pl.BlockSpec((_BN, DIM), lambda j, i: (i, 0)),
        ],
        out_specs=[
            pl.BlockSpec((_BN, DIM), lambda j, i: (i, 0)),
            pl.BlockSpec((1, _BN, DP), lambda j, i: (j, i, 0)),
        ],
        out_shape=[
            jax.ShapeDtypeStruct((N_NODES, DIM), jnp.float32),
            jax.ShapeDtypeStruct((2, N_NODES, DP), jnp.float32),
        ],
    )(hist_t, emb)


def _scale_body(t_ref, dinv_ref, z_ref):
    j = pl.program_id(0)
    d = dinv_ref[...]
    z_ref[...] = _z2_block(d * d * t_ref[...], j)


def _tc_scale(t_cat, dinv_rep):
    return pl.pallas_call(
        _scale_body,
        grid=(2, N_NODES // _BN),
        in_specs=[
            pl.BlockSpec((_BN, DIM), lambda j, i: (i, 0)),
            pl.BlockSpec((_BN, DIM), lambda j, i: (i, 0)),
        ],
        out_specs=pl.BlockSpec((1, _BN, DP), lambda j, i: (j, i, 0)),
        out_shape=jax.ShapeDtypeStruct((2, N_NODES, DP), jnp.float32),
    )(t_cat, dinv_rep)


def _final_body(t0_ref, t1_ref, t2_ref, dinv_ref, emb_ref, out_ref):
    tsum = t0_ref[...] + t1_ref[...] + t2_ref[...]
    out_ref[...] = (emb_ref[...] + dinv_ref[...] * tsum) * 0.25


def _tc_final(t0, t1, t2, dinv_rep, emb):
    spec = pl.BlockSpec((_BN, DIM), lambda i: (i, 0))
    return pl.pallas_call(
        _final_body,
        grid=(N_NODES // _BN,),
        in_specs=[spec] * 5,
        out_specs=spec,
        out_shape=jax.ShapeDtypeStruct((N_NODES, DIM), jnp.float32),
    )(t0, t1, t2, dinv_rep, emb)


def _t_cat(t2):
    return t2[:, :QN, :].reshape(NH, QN * 2, DIM).reshape(N_NODES, DIM)


def kernel(edge_index, emb_weight):
    src = edge_index[0].astype(jnp.int32)
    dst = edge_index[1].astype(jnp.int32)

    dst_pad = jnp.concatenate(
        [dst, jnp.full((E_PAD - N_EDGES,), N_NODES, jnp.int32)])
    src_r = ((dst & 1) * N_NODES + src).reshape(ROWS, SUB)
    hs = jnp.arange(NH, dtype=jnp.int32)[:, None] * HN
    local = dst[None, :] - hs
    dstq = jnp.where((local >= 0) & (local < HN), local >> 1,
                     jnp.int32(TRASH)).reshape(NH, ROWS, SUB)
    zero_tile = jnp.zeros((ZC, DP), jnp.float32)

    hist = _deg_kernel(dst_pad)
    hist_t = hist.T[:N_NODES]
    dinv_rep, z = _tc_finalize(hist_t, emb_weight)

    zf = z.reshape(2 * N_NODES, DP)
    t0 = _layer_kernel(src_r, dstq, zf, zero_tile)
    t0c = _t_cat(t0)
    zf = _tc_scale(t0c, dinv_rep).reshape(2 * N_NODES, DP)
    t1 = _layer_kernel(src_r, dstq, zf, zero_tile)
    t1c = _t_cat(t1)
    zf = _tc_scale(t1c, dinv_rep).reshape(2 * N_NODES, DP)
    t2 = _layer_kernel(src_r, dstq, zf, zero_tile)

    out = _tc_final(t0c, t1c, _t_cat(t2), dinv_rep, emb_weight)
    return (emb_weight, out)


# speedup vs baseline: 7.8324x; 1.0179x over previous
import functools

import jax
import jax.numpy as jnp
from jax import lax
from jax.experimental import pallas as pl
from jax.experimental.pallas import tpu as pltpu
from jax.experimental.pallas import tpu_sc as plsc

N_NODES = 50000
DIM = 64
DP = 128
N_EDGES = 800000

NC = 2
NS = 16
NH = 2
HN = N_NODES // NH
QN = HN // 2
QROWS = 12544
QPT = QROWS // NS
TRASH = QN
ZC = 392

SUB = 50
ROWS = N_EDGES // SUB
RPT = ROWS // NS
GRP = 4
CH8 = 8
NCH = RPT // CH8

E_PAD = 800256
EPT = E_PAD // (NC * NS)
ECH = 8336
HIST = 51200
NW = NC * NS

_mesh = plsc.VectorSubcoreMesh(core_axis_name="c", subcore_axis_name="s")
_cp = pltpu.CompilerParams(needs_layout_passes=False)


@functools.partial(
    pl.kernel,
    out_type=jax.ShapeDtypeStruct((NW, HIST), jnp.float32),
    mesh=_mesh,
    scratch_types=[
        pltpu.VMEM((ECH,), jnp.int32),
        pltpu.VMEM((HIST,), jnp.float32),
    ],
    compiler_params=_cp,
)
def _deg_kernel(dst_hbm, hist_hbm, idx_v, hist_v):
    c = lax.axis_index("c")
    s = lax.axis_index("s")
    w = s * NC + c

    def zero_body(i, _):
        hist_v[pl.ds(i * 16, 16)] = jnp.zeros((16,), jnp.float32)
        return 0
    lax.fori_loop(0, HIST // 16, zero_body, 0)

    ones16 = jnp.ones((16,), jnp.float32)

    def chunk_body(k, _):
        pltpu.sync_copy(dst_hbm.at[pl.ds(w * EPT + k * ECH, ECH)], idx_v)

        def scat_body(i, _2):
            idx16 = idx_v[pl.ds(i * 16, 16)]
            plsc.addupdate_scatter(hist_v, [idx16], ones16)
            return 0
        lax.fori_loop(0, ECH // 16, scat_body, 0)
        return 0
    lax.fori_loop(0, EPT // ECH, chunk_body, 0)

    pltpu.sync_copy(hist_v, hist_hbm.at[w])


@functools.partial(
    pl.kernel,
    out_type=jax.ShapeDtypeStruct((NH, QROWS, DP), jnp.float32),
    mesh=_mesh,
    scratch_types=[
        pltpu.VMEM_SHARED((QROWS, DP), jnp.float32),
        pltpu.VMEM((CH8, SUB), jnp.int32),
        pltpu.VMEM((CH8, SUB), jnp.int32),
        pltpu.VMEM((GRP, SUB, DP), jnp.float32),
        [pltpu.SemaphoreType.DMA] * 4,
        [pltpu.SemaphoreType.DMA] * 4,
    ],
    compiler_params=_cp,
)
def _layer_kernel(src_hbm, dstq_hbm, z_hbm, zero_hbm, t_hbm,
                  acc_sp, src_v, dst_v, rows_v, sems_g, sems_s):
    c = lax.axis_index("c")
    s = lax.axis_index("s")

    for k in range(QPT // ZC):
        pltpu.sync_copy(zero_hbm,
                        acc_sp.at[pl.ds(s * QPT + k * ZC, ZC)])
    plsc.subcore_barrier()

    def chunk_body(ch, _):
        row0 = s * RPT + ch * CH8
        pltpu.sync_copy(src_hbm.at[pl.ds(row0, CH8)], src_v)
        pltpu.sync_copy(dstq_hbm.at[c, pl.ds(row0, CH8)], dst_v)

        def fg(b, r):
            pltpu.async_copy(z_hbm.at[src_v.at[r]], rows_v.at[b],
                             sems_g[b])

        def dg(b):
            pltpu.make_async_copy(
                z_hbm.at[src_v.at[0]], rows_v.at[0], sems_g[b]).wait()

        def fs(b, r):
            pltpu.async_copy(rows_v.at[b], acc_sp.at[dst_v.at[r]],
                             sems_s[b], add=True)

        def ds(b):
            pltpu.make_async_copy(
                rows_v.at[0], acc_sp.at[dst_v.at[0]], sems_s[b]).wait()

        for b in range(4):
            fg(b, b)
        for b in range(4):
            dg(b); fs(b, b)
        for b in range(4):
            ds(b); fg(b, 4 + b)
        for b in range(4):
            dg(b); fs(b, 4 + b)
        for b in range(4):
            ds(b)
        return 0

    lax.fori_loop(0, NCH, chunk_body, 0)
    plsc.subcore_barrier()

    for k in range(QPT // ZC):
        off = s * QPT + k * ZC
        pltpu.sync_copy(acc_sp.at[pl.ds(off, ZC)],
                        t_hbm.at[c, pl.ds(off, ZC)])


_BN = 2000


def _z2_block(z, j):
    zero = jnp.zeros((_BN, DIM), jnp.float32)
    top = jnp.concatenate([z, zero], axis=1)
    bot = jnp.concatenate([zero, z], axis=1)
    return jnp.where(j == 0, top, bot)[None]


def _finalize_body(hist_ref, emb_ref, dinv_ref, z0_ref):
    j = pl.program_id(0)
    deg = jnp.sum(hist_ref[...], axis=1, keepdims=True)
    safe = jnp.where(deg > 0, deg, 1.0)
    dinv = jnp.where(deg > 0, lax.rsqrt(safe), 0.0)
    dinv_rep = jnp.broadcast_to(dinv, (_BN, DIM))
    dinv_ref[...] = dinv_rep
    z0_ref[...] = _z2_block(dinv_rep * emb_ref[...], j)


def _tc_finalize(hist_t, emb):
    return pl.pallas_call(
        _finalize_body,
        grid=(2, N_NODES // _BN),
        in_specs=[
            pl.BlockSpec((_BN, NW), lambda j, i: (i, 0)),
            pl.BlockSpec((_BN, DIM), lambda j, i: (i, 0)),
        ],
        out_specs=[
            pl.BlockSpec((_BN, DIM), lambda j, i: (i, 0)),
            pl.BlockSpec((1, _BN, DP), lambda j, i: (j, i, 0)),
        ],
        out_shape=[
            jax.ShapeDtypeStruct((N_NODES, DIM), jnp.float32),
            jax.ShapeDtypeStruct((2, N_NODES, DP), jnp.float32),
        ],
    )(hist_t, emb)


def _scale_body(t_ref, dinv_ref, z_ref):
    j = pl.program_id(0)
    d = dinv_ref[...]
    z_ref[...] = _z2_block(d * d * t_ref[...], j)


def _tc_scale(t_cat, dinv_rep):
    return pl.pallas_call(
        _scale_body,
        grid=(2, N_NODES // _BN),
        in_specs=[
            pl.BlockSpec((_BN, DIM), lambda j, i: (i, 0)),
            pl.BlockSpec((_BN, DIM), lambda j, i: (i, 0)),
        ],
        out_specs=pl.BlockSpec((1, _BN, DP), lambda j, i: (j, i, 0)),
        out_shape=jax.ShapeDtypeStruct((2, N_NODES, DP), jnp.float32),
    )(t_cat, dinv_rep)


def _final_body(t0_ref, t1_ref, t2_ref, dinv_ref, emb_ref, out_ref):
    tsum = t0_ref[...] + t1_ref[...] + t2_ref[...]
    out_ref[...] = (emb_ref[...] + dinv_ref[...] * tsum) * 0.25


def _tc_final(t0, t1, t2, dinv_rep, emb):
    spec = pl.BlockSpec((_BN, DIM), lambda i: (i, 0))
    return pl.pallas_call(
        _final_body,
        grid=(N_NODES // _BN,),
        in_specs=[spec] * 5,
        out_specs=spec,
        out_shape=jax.ShapeDtypeStruct((N_NODES, DIM), jnp.float32),
    )(t0, t1, t2, dinv_rep, emb)


def _t_cat(t2):
    return t2[:, :QN, :].reshape(NH, QN * 2, DIM).reshape(N_NODES, DIM)


def kernel(edge_index, emb_weight):
    src = edge_index[0].astype(jnp.int32)
    dst = edge_index[1].astype(jnp.int32)

    dst_pad = jnp.concatenate(
        [dst, jnp.full((E_PAD - N_EDGES,), N_NODES, jnp.int32)])
    src_r = ((dst & 1) * N_NODES + src).reshape(ROWS, SUB)
    hs = jnp.arange(NH, dtype=jnp.int32)[:, None] * HN
    local = dst[None, :] - hs
    dstq = jnp.where((local >= 0) & (local < HN), local >> 1,
                     jnp.int32(TRASH)).reshape(NH, ROWS, SUB)
    zero_tile = jnp.zeros((ZC, DP), jnp.float32)

    hist = _deg_kernel(dst_pad)
    hist_t = hist.T[:N_NODES]
    dinv_rep, z = _tc_finalize(hist_t, emb_weight)

    zf = z.reshape(2 * N_NODES, DP)
    t0 = _layer_kernel(src_r, dstq, zf, zero_tile)
    t0c = _t_cat(t0)
    zf = _tc_scale(t0c, dinv_rep).reshape(2 * N_NODES, DP)
    t1 = _layer_kernel(src_r, dstq, zf, zero_tile)
    t1c = _t_cat(t1)
    zf = _tc_scale(t1c, dinv_rep).reshape(2 * N_NODES, DP)
    t2 = _layer_kernel(src_r, dstq, zf, zero_tile)

    out = _tc_final(t0c, t1c, _t_cat(t2), dinv_rep, emb_weight)
    return (emb_weight, out)
